# baseline (speedup 1.0000x reference)
import jax
import jax.numpy as jnp
from jax import lax
from jax.experimental.layout import Format, Layout
from jax.experimental import pallas as pl
from jax.experimental.pallas import tpu as pltpu
from jax.experimental.pallas import tpu_sc as plsc

GLOBAL_MEAN = 3.5
D = 32
NC = 2
NS = 16
NW = NC * NS
L = 16
CE = 32
NCH = 16
GR = D * CE

N_P = 100000
N_Q = 1000000
NT_P = N_P // 128
NT_Q = N_Q // 128
PTH = NT_P * 128
QTH = NT_Q * 128
PTB = N_P - 160
QTB = N_Q - 128


def _granule_base(iv, th):
    ic = jnp.minimum(iv, th - 1)
    gi = lax.shift_left(lax.shift_right_logical(ic, 7), 6) + \
        jnp.bitwise_and(lax.shift_right_logical(ic, 4), 7)
    return gi, jnp.bitwise_and(ic, 15)


def _mf_body(sid_hbm, pid_hbm, pv0, qv0, pt_hbm, qt_hbm,
             sb_hbm, pb_hbm, out_hbm,
             sid_v, pid_v, pidx, qidx, ptidx, qtidx,
             pdst, qdst, ptd, qtd, bs_v, bp_v, out_v,
             semb, semt, semp0, semp1, semq0, semq1):
    b_per_w = sid_v.shape[0]
    wid = lax.axis_index("s") * NC + lax.axis_index("c")
    base = wid * b_per_w
    semp = (semp0, semp1)
    semq = (semq0, semq1)
    pviews = (pv0,)
    qviews = (qv0,)
    lane = lax.iota(jnp.int32, L)

    pltpu.sync_copy(sid_hbm.at[pl.ds(base, b_per_w)], sid_v)
    pltpu.sync_copy(pid_hbm.at[pl.ds(base, b_per_w)], pid_v)

    bias_copies = []
    for k in range(b_per_w // 128):
        sl = pl.ds(k * 128, 128)
        bias_copies.append(pltpu.async_copy(sb_hbm.at[sid_v.at[sl]],
                                            bs_v.at[sl], semb))
        bias_copies.append(pltpu.async_copy(pb_hbm.at[pid_v.at[sl]],
                                            bp_v.at[sl], semb))

    def fire_one(c, buf, ids_v, views, idx, dst, sem, th, nt):
        for sub in range(0, CE, L):
            iv = ids_v[pl.ds(c * CE + sub, L)]
            gi, _ = _granule_base(iv, th)
            for d in range(D):
                cd = (d >> 3) * nt * 64 + (d & 7) * 8
                idx[pl.ds(buf * GR + d * CE + sub, L)] = gi + cd
        for s in range(GR // 128):
            pltpu.async_copy(
                views[0].at[idx.at[pl.ds(buf * GR + s * 128, 128)]],
                dst.at[pl.ds(buf * GR + s * 128, 128)], sem[buf])

    def fire(c, buf):
        fire_one(c, buf, pid_v, qviews, qidx, qdst, semq, QTH, NT_Q)
        fire_one(c, buf, sid_v, pviews, pidx, pdst, semp, PTH, NT_P)

    def drain(buf):
        pltpu.make_async_copy(qv0.at[pl.ds(0, GR)],
                              qdst.at[pl.ds(buf * GR, GR)],
                              semq[buf]).wait()
        pltpu.make_async_copy(pv0.at[pl.ds(0, GR)],
                              pdst.at[pl.ds(buf * GR, GR)],
                              semp[buf]).wait()

    def patch_tail(buf, ids_v, e0, sub, tidx, dst, td, tail, th, tb):
        iv = ids_v[pl.ds(e0 + sub, L)]
        it = iv >= th
        ntail = plsc.all_reduce_population_count(it)

        @pl.when(ntail[0] > 0)
        def _():
            fallback = sub + lane
            mi = 1 + lax.shift_right_arithmetic(iv - th, 31)
            tidx[pl.ds(sub, L)] = fallback + mi * (
                lax.shift_right_logical(iv - tb, 2) - fallback)
            pltpu.async_copy(
                tail.at[tidx.at[pl.ds(sub, L)]],
                td.at[pl.ds(sub, L)], semt).wait()
            _, low = _granule_base(iv, th)
            off = lax.shift_left(jnp.bitwise_and(iv - tb, 3), 5)
            for d in range(D):
                rowv = buf * GR + d * CE + sub + lane
                tv = plsc.load_gather(td, [sub + lane, off + d], mask=it)
                plsc.store_scatter(dst, [rowv, low], tv, mask=it)

    def compute(c, buf):
        for sub in range(0, CE, L):
            e0 = c * CE
            patch_tail(buf, sid_v, e0, sub, ptidx, pdst, ptd, pt_hbm,
                       PTH, PTB)
            patch_tail(buf, pid_v, e0, sub, qtidx, qdst, qtd, qt_hbm,
                       QTH, QTB)
            _, lows = _granule_base(sid_v[pl.ds(e0 + sub, L)], PTH)
            _, lowq = _granule_base(pid_v[pl.ds(e0 + sub, L)], QTH)
            sl = pl.ds(e0 + sub, L)
            acc = bs_v[sl] + bp_v[sl] + GLOBAL_MEAN
            for d in range(D):
                rowv = buf * GR + d * CE + sub + lane
                acc = acc + (plsc.load_gather(pdst, [rowv, lows])
                             * plsc.load_gather(qdst, [rowv, lowq]))
            out_v[sl] = acc

    fire(0, 0)
    for h in bias_copies:
        h.wait()

    def step(k, carry):
        c0 = 2 * k
        fire(c0 + 1, 1)
        drain(0)
        compute(c0, 0)

        @pl.when(c0 + 2 < NCH)
        def _():
            fire(c0 + 2, 0)

        drain(1)
        compute(c0 + 1, 1)
        return carry

    lax.fori_loop(0, NCH // 2, step, 0)
    pltpu.sync_copy(out_v, out_hbm.at[pl.ds(base, b_per_w)])


def _granule_view(T, nt):
    th = nt * 128
    return (T[:th].reshape(nt, 8, 16, 4, 8).transpose(3, 0, 4, 1, 2)
            .reshape(-1, 16))


@jax.jit
def kernel(SIDs, PIDs, P, Q, scientist_bias, paper_bias):
    B = SIDs.shape[0]
    b_per_w = B // NW
    sids = SIDs.astype(jnp.int32)
    pids = PIDs.astype(jnp.int32)
    pv = _granule_view(P, NT_P)
    qv = _granule_view(Q, NT_Q)
    pt = P[PTB:].reshape(-1, 128)
    qt = Q[QTB:].reshape(-1, 128)
    sb = scientist_bias.reshape(-1)
    pb = paper_bias.reshape(-1)

    mesh = plsc.VectorSubcoreMesh(core_axis_name="c", subcore_axis_name="s")
    f = pl.kernel(
        _mf_body,
        out_type=jax.ShapeDtypeStruct((B,), jnp.float32),
        mesh=mesh,
        compiler_params=pltpu.CompilerParams(
            needs_layout_passes=False, use_tc_tiling_on_sc=False),
        scratch_types=[
            pltpu.VMEM((b_per_w,), jnp.int32),
            pltpu.VMEM((b_per_w,), jnp.int32),
            pltpu.VMEM((2 * GR,), jnp.int32),
            pltpu.VMEM((2 * GR,), jnp.int32),
            pltpu.VMEM((CE,), jnp.int32),
            pltpu.VMEM((CE,), jnp.int32),
            pltpu.VMEM((2 * GR, 16), jnp.float32),
            pltpu.VMEM((2 * GR, 16), jnp.float32),
            pltpu.VMEM((CE, 128), jnp.float32),
            pltpu.VMEM((CE, 128), jnp.float32),
            pltpu.VMEM((b_per_w,), jnp.float32),
            pltpu.VMEM((b_per_w,), jnp.float32),
            pltpu.VMEM((b_per_w,), jnp.float32),
            pltpu.SemaphoreType.DMA,
            pltpu.SemaphoreType.DMA,
            pltpu.SemaphoreType.DMA,
            pltpu.SemaphoreType.DMA,
            pltpu.SemaphoreType.DMA,
            pltpu.SemaphoreType.DMA,
        ],
    )
    return f(sids, pids, pv, qv, pt, qt, sb, pb)


# speedup vs baseline: 13.2907x; 13.2907x over previous
import jax
import jax.numpy as jnp
from jax import lax
from jax.experimental import pallas as pl
from jax.experimental.pallas import tpu as pltpu
from jax.experimental.pallas import tpu_sc as plsc

GLOBAL_MEAN = 3.5
D = 32
NC = 2
NS = 16
NW = NC * NS
L = 16
CE = 32
NCH = 16
GR = D * CE

N_P = 100000
N_Q = 1000000
NT_P = N_P // 128
NT_Q = N_Q // 128
PTH = NT_P * 128
QTH = NT_Q * 128
PTB = N_P - 160
QTB = N_Q - 128


def _granule_base(iv, th):
    ic = jnp.minimum(iv, th - 1)
    gi = lax.shift_left(lax.shift_right_logical(ic, 7), 6) + \
        jnp.bitwise_and(lax.shift_right_logical(ic, 4), 7)
    return gi, jnp.bitwise_and(ic, 15)


def _mf_body(sid_hbm, pid_hbm, pv0, qv0, pt_hbm, qt_hbm,
             sb_hbm, pb_hbm, out_hbm,
             sid_v, pid_v, pidx, qidx, ptidx, qtidx,
             pdst, qdst, ptd, qtd, bs_v, bp_v, out_v,
             semb, semt, semp0, semp1, semq0, semq1):
    b_per_w = sid_v.shape[0]
    wid = lax.axis_index("s") * NC + lax.axis_index("c")
    base = wid * b_per_w
    semp = (semp0, semp1)
    semq = (semq0, semq1)
    pviews = (pv0,)
    qviews = (qv0,)
    lane = lax.iota(jnp.int32, L)

    pltpu.sync_copy(sid_hbm.at[pl.ds(base, b_per_w)], sid_v)
    pltpu.sync_copy(pid_hbm.at[pl.ds(base, b_per_w)], pid_v)

    bias_copies = []
    for k in range(b_per_w // 128):
        sl = pl.ds(k * 128, 128)
        bias_copies.append(pltpu.async_copy(sb_hbm.at[sid_v.at[sl]],
                                            bs_v.at[sl], semb))
        bias_copies.append(pltpu.async_copy(pb_hbm.at[pid_v.at[sl]],
                                            bp_v.at[sl], semb))

    def fire_one(c, buf, ids_v, views, idx, dst, sem, th, nt):
        for sub in range(0, CE, L):
            iv = ids_v[pl.ds(c * CE + sub, L)]
            gi, _ = _granule_base(iv, th)
            for d in range(D):
                cd = (d >> 3) * nt * 64 + (d & 7) * 8
                idx[pl.ds(buf * GR + d * CE + sub, L)] = gi + cd
        for s in range(GR // 128):
            pltpu.async_copy(
                views[0].at[idx.at[pl.ds(buf * GR + s * 128, 128)]],
                dst.at[pl.ds(buf * GR + s * 128, 128)], sem[buf])

    def fire(c, buf):
        fire_one(c, buf, pid_v, qviews, qidx, qdst, semq, QTH, NT_Q)
        fire_one(c, buf, sid_v, pviews, pidx, pdst, semp, PTH, NT_P)

    def drain(buf):
        pltpu.make_async_copy(qv0.at[pl.ds(0, GR)],
                              qdst.at[pl.ds(buf * GR, GR)],
                              semq[buf]).wait()
        pltpu.make_async_copy(pv0.at[pl.ds(0, GR)],
                              pdst.at[pl.ds(buf * GR, GR)],
                              semp[buf]).wait()

    def patch_tail(buf, ids_v, e0, sub, tidx, dst, td, tail, th, tb):
        iv = ids_v[pl.ds(e0 + sub, L)]
        it = iv >= th
        ntail = plsc.all_reduce_population_count(it)

        @pl.when(ntail[0] > 0)
        def _():
            fallback = sub + lane
            mi = 1 + lax.shift_right_arithmetic(iv - th, 31)
            tidx[pl.ds(sub, L)] = fallback + mi * (
                lax.shift_right_logical(iv - tb, 2) - fallback)
            pltpu.async_copy(
                tail.at[tidx.at[pl.ds(sub, L)]],
                td.at[pl.ds(sub, L)], semt).wait()
            _, low = _granule_base(iv, th)
            off = lax.shift_left(jnp.bitwise_and(iv - tb, 3), 5)
            for d in range(D):
                rowv = buf * GR + d * CE + sub + lane
                tv = plsc.load_gather(td, [sub + lane, off + d], mask=it)
                plsc.store_scatter(dst, [rowv, low], tv, mask=it)

    def compute(c, buf):
        for sub in range(0, CE, L):
            e0 = c * CE
            patch_tail(buf, sid_v, e0, sub, ptidx, pdst, ptd, pt_hbm,
                       PTH, PTB)
            patch_tail(buf, pid_v, e0, sub, qtidx, qdst, qtd, qt_hbm,
                       QTH, QTB)
            _, lows = _granule_base(sid_v[pl.ds(e0 + sub, L)], PTH)
            _, lowq = _granule_base(pid_v[pl.ds(e0 + sub, L)], QTH)
            sl = pl.ds(e0 + sub, L)
            acc = bs_v[sl] + bp_v[sl] + GLOBAL_MEAN
            for d in range(D):
                rowv = buf * GR + d * CE + sub + lane
                acc = acc + (plsc.load_gather(pdst, [rowv, lows])
                             * plsc.load_gather(qdst, [rowv, lowq]))
            out_v[sl] = acc

    fire(0, 0)
    for h in bias_copies:
        h.wait()

    def step(k, carry):
        c0 = 2 * k
        fire(c0 + 1, 1)
        drain(0)
        compute(c0, 0)

        @pl.when(c0 + 2 < NCH)
        def _():
            fire(c0 + 2, 0)

        drain(1)
        compute(c0 + 1, 1)
        return carry

    lax.fori_loop(0, NCH // 2, step, 0)
    pltpu.sync_copy(out_v, out_hbm.at[pl.ds(base, b_per_w)])


def _granule_view(T, nt):
    th = nt * 128
    return (T[:th].T.reshape(4, 8, nt, 128).transpose(0, 2, 1, 3)
            .reshape(-1, 16))


@jax.jit
def kernel(SIDs, PIDs, P, Q, scientist_bias, paper_bias):
    B = SIDs.shape[0]
    b_per_w = B // NW
    sids = SIDs.astype(jnp.int32)
    pids = PIDs.astype(jnp.int32)
    pv = _granule_view(P, NT_P)
    qv = _granule_view(Q, NT_Q)
    pt = P[PTB:].reshape(-1, 128)
    qt = Q[QTB:].reshape(-1, 128)
    sb = scientist_bias.reshape(-1)
    pb = paper_bias.reshape(-1)

    mesh = plsc.VectorSubcoreMesh(core_axis_name="c", subcore_axis_name="s")
    f = pl.kernel(
        _mf_body,
        out_type=jax.ShapeDtypeStruct((B,), jnp.float32),
        mesh=mesh,
        compiler_params=pltpu.CompilerParams(
            needs_layout_passes=False, use_tc_tiling_on_sc=False),
        scratch_types=[
            pltpu.VMEM((b_per_w,), jnp.int32),
            pltpu.VMEM((b_per_w,), jnp.int32),
            pltpu.VMEM((2 * GR,), jnp.int32),
            pltpu.VMEM((2 * GR,), jnp.int32),
            pltpu.VMEM((CE,), jnp.int32),
            pltpu.VMEM((CE,), jnp.int32),
            pltpu.VMEM((2 * GR, 16), jnp.float32),
            pltpu.VMEM((2 * GR, 16), jnp.float32),
            pltpu.VMEM((CE, 128), jnp.float32),
            pltpu.VMEM((CE, 128), jnp.float32),
            pltpu.VMEM((b_per_w,), jnp.float32),
            pltpu.VMEM((b_per_w,), jnp.float32),
            pltpu.VMEM((b_per_w,), jnp.float32),
            pltpu.SemaphoreType.DMA,
            pltpu.SemaphoreType.DMA,
            pltpu.SemaphoreType.DMA,
            pltpu.SemaphoreType.DMA,
            pltpu.SemaphoreType.DMA,
            pltpu.SemaphoreType.DMA,
        ],
    )
    return f(sids, pids, pv, qv, pt, qt, sb, pb)


# speedup vs baseline: 13.3009x; 1.0008x over previous
import jax
import jax.numpy as jnp
from jax import lax
from jax.experimental import pallas as pl
from jax.experimental.pallas import tpu as pltpu
from jax.experimental.pallas import tpu_sc as plsc

GLOBAL_MEAN = 3.5
D = 32
NC = 2
NS = 16
NW = NC * NS
L = 16
CE = 32
NCH = 16
GR = D * CE

N_P = 100000
N_Q = 1000000
NT_P = N_P // 128
NT_Q = N_Q // 128
PTH = NT_P * 128
QTH = NT_Q * 128
PTB = N_P - 160
QTB = N_Q - 128


def _granule_base(iv, th):
    ic = jnp.minimum(iv, th - 1)
    gi = lax.shift_left(lax.shift_right_logical(ic, 7), 6) + \
        jnp.bitwise_and(lax.shift_right_logical(ic, 4), 7)
    return gi, jnp.bitwise_and(ic, 15)


def _mf_body(sid_hbm, pid_hbm, pv0, qv0, pt_hbm, qt_hbm,
             sb_hbm, pb_hbm, out_hbm,
             sid_v, pid_v, pidx, qidx, ptidx, qtidx,
             pdst, qdst, ptd, qtd, bs_v, bp_v, out_v,
             semb, semt, semp0, semp1, semq0, semq1):
    b_per_w = sid_v.shape[0]
    wid = lax.axis_index("s") * NC + lax.axis_index("c")
    base = wid * b_per_w
    semp = (semp0, semp1)
    semq = (semq0, semq1)
    pviews = (pv0,)
    qviews = (qv0,)
    lane = lax.iota(jnp.int32, L)

    pltpu.sync_copy(sid_hbm.at[pl.ds(base, b_per_w)], sid_v)
    pltpu.sync_copy(pid_hbm.at[pl.ds(base, b_per_w)], pid_v)

    bias_copies = []
    for k in range(b_per_w // 128):
        sl = pl.ds(k * 128, 128)
        bias_copies.append(pltpu.async_copy(sb_hbm.at[sid_v.at[sl]],
                                            bs_v.at[sl], semb))
        bias_copies.append(pltpu.async_copy(pb_hbm.at[pid_v.at[sl]],
                                            bp_v.at[sl], semb))

    def fire_one(c, buf, ids_v, views, idx, dst, sem, th, nt):
        for sub in range(0, CE, L):
            iv = ids_v[pl.ds(c * CE + sub, L)]
            gi, _ = _granule_base(iv, th)
            for d in range(D):
                cd = (d >> 3) * nt * 64 + (d & 7) * 8
                idx[pl.ds(buf * GR + d * CE + sub, L)] = gi + cd
        for s in range(GR // 128):
            pltpu.async_copy(
                views[0].at[idx.at[pl.ds(buf * GR + s * 128, 128)]],
                dst.at[pl.ds(buf * GR + s * 128, 128)], sem[buf])

    def fire(c, buf):
        fire_one(c, buf, pid_v, qviews, qidx, qdst, semq, QTH, NT_Q)
        fire_one(c, buf, sid_v, pviews, pidx, pdst, semp, PTH, NT_P)

    def drain(buf):
        pltpu.make_async_copy(qv0.at[pl.ds(0, GR)],
                              qdst.at[pl.ds(buf * GR, GR)],
                              semq[buf]).wait()
        pltpu.make_async_copy(pv0.at[pl.ds(0, GR)],
                              pdst.at[pl.ds(buf * GR, GR)],
                              semp[buf]).wait()

    def patch_tail(buf, ids_v, e0, sub, tidx, dst, td, tail, th, tb):
        iv = ids_v[pl.ds(e0 + sub, L)]
        it = iv >= th
        ntail = plsc.all_reduce_population_count(it)

        @pl.when(ntail[0] > 0)
        def _():
            fallback = sub + lane
            mi = 1 + lax.shift_right_arithmetic(iv - th, 31)
            tidx[pl.ds(sub, L)] = fallback + mi * (
                lax.shift_right_logical(iv - tb, 2) - fallback)
            pltpu.async_copy(
                tail.at[tidx.at[pl.ds(sub, L)]],
                td.at[pl.ds(sub, L)], semt).wait()
            _, low = _granule_base(iv, th)
            off = lax.shift_left(jnp.bitwise_and(iv - tb, 3), 5)
            for d in range(D):
                rowv = buf * GR + d * CE + sub + lane
                tv = plsc.load_gather(td, [sub + lane, off + d], mask=it)
                plsc.store_scatter(dst, [rowv, low], tv, mask=it)

    def compute(c, buf):
        for sub in range(0, CE, L):
            e0 = c * CE
            patch_tail(buf, sid_v, e0, sub, ptidx, pdst, ptd, pt_hbm,
                       PTH, PTB)
            patch_tail(buf, pid_v, e0, sub, qtidx, qdst, qtd, qt_hbm,
                       QTH, QTB)
            _, lows = _granule_base(sid_v[pl.ds(e0 + sub, L)], PTH)
            _, lowq = _granule_base(pid_v[pl.ds(e0 + sub, L)], QTH)
            sl = pl.ds(e0 + sub, L)
            acc = bs_v[sl] + bp_v[sl] + GLOBAL_MEAN
            for d in range(D):
                rowv = buf * GR + d * CE + sub + lane
                acc = acc + (plsc.load_gather(pdst, [rowv, lows])
                             * plsc.load_gather(qdst, [rowv, lowq]))
            out_v[sl] = acc

    fire(0, 0)
    for h in bias_copies:
        h.wait()

    def step(k, carry):
        c0 = 2 * k
        fire(c0 + 1, 1)
        drain(0)
        compute(c0, 0)

        @pl.when(c0 + 2 < NCH)
        def _():
            fire(c0 + 2, 0)

        drain(1)
        compute(c0 + 1, 1)
        return carry

    lax.fori_loop(0, NCH // 2, step, 0)
    pltpu.sync_copy(out_v, out_hbm.at[pl.ds(base, b_per_w)])


def _granule_view(T, nt):
    from jax.experimental.layout import Format, Layout
    th = nt * 128
    x = jax.device_put(
        T[:th].T.reshape(4, 8, nt, 128),
        Format(Layout(major_to_minor=(0, 2, 1, 3), tiling=((8, 128),)),
               jax.sharding.SingleDeviceSharding(jax.devices()[0])))
    return x.transpose(0, 2, 1, 3).reshape(-1, 16)


@jax.jit
def kernel(SIDs, PIDs, P, Q, scientist_bias, paper_bias):
    B = SIDs.shape[0]
    b_per_w = B // NW
    sids = SIDs.astype(jnp.int32)
    pids = PIDs.astype(jnp.int32)
    pv = _granule_view(P, NT_P)
    qv = _granule_view(Q, NT_Q)
    pt = P[PTB:].reshape(-1, 128)
    qt = Q[QTB:].reshape(-1, 128)
    sb = scientist_bias.reshape(-1)
    pb = paper_bias.reshape(-1)

    mesh = plsc.VectorSubcoreMesh(core_axis_name="c", subcore_axis_name="s")
    f = pl.kernel(
        _mf_body,
        out_type=jax.ShapeDtypeStruct((B,), jnp.float32),
        mesh=mesh,
        compiler_params=pltpu.CompilerParams(
            needs_layout_passes=False, use_tc_tiling_on_sc=False),
        scratch_types=[
            pltpu.VMEM((b_per_w,), jnp.int32),
            pltpu.VMEM((b_per_w,), jnp.int32),
            pltpu.VMEM((2 * GR,), jnp.int32),
            pltpu.VMEM((2 * GR,), jnp.int32),
            pltpu.VMEM((CE,), jnp.int32),
            pltpu.VMEM((CE,), jnp.int32),
            pltpu.VMEM((2 * GR, 16), jnp.float32),
            pltpu.VMEM((2 * GR, 16), jnp.float32),
            pltpu.VMEM((CE, 128), jnp.float32),
            pltpu.VMEM((CE, 128), jnp.float32),
            pltpu.VMEM((b_per_w,), jnp.float32),
            pltpu.VMEM((b_per_w,), jnp.float32),
            pltpu.VMEM((b_per_w,), jnp.float32),
            pltpu.SemaphoreType.DMA,
            pltpu.SemaphoreType.DMA,
            pltpu.SemaphoreType.DMA,
            pltpu.SemaphoreType.DMA,
            pltpu.SemaphoreType.DMA,
            pltpu.SemaphoreType.DMA,
        ],
    )
    return f(sids, pids, pv, qv, pt, qt, sb, pb)


# speedup vs baseline: 13.3055x; 1.0003x over previous
import jax
import jax.numpy as jnp
from jax import lax
from jax.experimental import pallas as pl
from jax.experimental.pallas import tpu as pltpu
from jax.experimental.pallas import tpu_sc as plsc

GLOBAL_MEAN = 3.5
D = 32
NC = 2
NS = 16
NW = NC * NS
L = 16
CE = 32
NCH = 16
GR = D * CE

N_P = 100000
N_Q = 1000000
NT_P = N_P // 128
NT_Q = N_Q // 128
PTH = NT_P * 128
QTH = NT_Q * 128
PTB = N_P - 160
QTB = N_Q - 128


def _granule_base(iv, th):
    ic = jnp.minimum(iv, th - 1)
    gi = lax.shift_left(lax.shift_right_logical(ic, 7), 6) + \
        jnp.bitwise_and(lax.shift_right_logical(ic, 4), 7)
    return gi, jnp.bitwise_and(ic, 15)


def _mf_body(sid_hbm, pid_hbm, pv0, qv0, pt_hbm, qt_hbm,
             sb_hbm, pb_hbm, out_hbm,
             sid_v, pid_v, pidx, qidx, ptidx, qtidx,
             pdst, qdst, ptd, qtd, bs_v, bp_v, out_v,
             semb, semt, semp0, semp1, semq0, semq1):
    b_per_w = sid_v.shape[0]
    wid = lax.axis_index("s") * NC + lax.axis_index("c")
    base = wid * b_per_w
    semp = (semp0, semp1)
    semq = (semq0, semq1)
    pviews = (pv0,)
    qviews = (qv0,)
    lane = lax.iota(jnp.int32, L)

    pltpu.sync_copy(sid_hbm.at[pl.ds(base, b_per_w)], sid_v)
    pltpu.sync_copy(pid_hbm.at[pl.ds(base, b_per_w)], pid_v)

    bias_copies = []
    for k in range(b_per_w // 128):
        sl = pl.ds(k * 128, 128)
        bias_copies.append(pltpu.async_copy(sb_hbm.at[sid_v.at[sl]],
                                            bs_v.at[sl], semb))
        bias_copies.append(pltpu.async_copy(pb_hbm.at[pid_v.at[sl]],
                                            bp_v.at[sl], semb))

    def fire_one(c, buf, ids_v, views, idx, dst, sem, th, nt):
        for sub in range(0, CE, L):
            iv = ids_v[pl.ds(c * CE + sub, L)]
            gi, _ = _granule_base(iv, th)
            for d in range(D):
                cd = (d >> 3) * nt * 64 + (d & 7) * 8
                idx[pl.ds(buf * GR + d * CE + sub, L)] = gi + cd
        for s in range(GR // 128):
            pltpu.async_copy(
                views[0].at[idx.at[pl.ds(buf * GR + s * 128, 128)]],
                dst.at[pl.ds(buf * GR + s * 128, 128)], sem[buf])

    def fire(c, buf):
        fire_one(c, buf, pid_v, qviews, qidx, qdst, semq, QTH, NT_Q)
        fire_one(c, buf, sid_v, pviews, pidx, pdst, semp, PTH, NT_P)

    def drain(buf):
        pltpu.make_async_copy(qv0.at[pl.ds(0, GR)],
                              qdst.at[pl.ds(buf * GR, GR)],
                              semq[buf]).wait()
        pltpu.make_async_copy(pv0.at[pl.ds(0, GR)],
                              pdst.at[pl.ds(buf * GR, GR)],
                              semp[buf]).wait()

    def patch_tail(buf, ids_v, e0, sub, tidx, dst, td, tail, th, tb):
        iv = ids_v[pl.ds(e0 + sub, L)]
        it = iv >= th
        ntail = plsc.all_reduce_population_count(it)

        @pl.when(ntail[0] > 0)
        def _():
            fallback = sub + lane
            mi = 1 + lax.shift_right_arithmetic(iv - th, 31)
            tidx[pl.ds(sub, L)] = fallback + mi * (
                lax.shift_right_logical(iv - tb, 2) - fallback)
            pltpu.async_copy(
                tail.at[tidx.at[pl.ds(sub, L)]],
                td.at[pl.ds(sub, L)], semt).wait()
            _, low = _granule_base(iv, th)
            off = lax.shift_left(jnp.bitwise_and(iv - tb, 3), 5)
            for d in range(D):
                rowv = buf * GR + d * CE + sub + lane
                tv = plsc.load_gather(td, [sub + lane, off + d], mask=it)
                plsc.store_scatter(dst, [rowv, low], tv, mask=it)

    def compute(c, buf):
        for sub in range(0, CE, L):
            e0 = c * CE
            patch_tail(buf, sid_v, e0, sub, ptidx, pdst, ptd, pt_hbm,
                       PTH, PTB)
            patch_tail(buf, pid_v, e0, sub, qtidx, qdst, qtd, qt_hbm,
                       QTH, QTB)
            _, lows = _granule_base(sid_v[pl.ds(e0 + sub, L)], PTH)
            _, lowq = _granule_base(pid_v[pl.ds(e0 + sub, L)], QTH)
            sl = pl.ds(e0 + sub, L)
            acc = bs_v[sl] + bp_v[sl] + GLOBAL_MEAN
            for d in range(D):
                rowv = buf * GR + d * CE + sub + lane
                acc = acc + (plsc.load_gather(pdst, [rowv, lows])
                             * plsc.load_gather(qdst, [rowv, lowq]))
            out_v[sl] = acc

    fire(0, 0)
    for h in bias_copies:
        h.wait()

    def step(k, carry):
        c0 = 2 * k
        fire(c0 + 1, 1)
        drain(0)
        compute(c0, 0)

        @pl.when(c0 + 2 < NCH)
        def _():
            fire(c0 + 2, 0)

        drain(1)
        compute(c0 + 1, 1)
        return carry

    lax.fori_loop(0, NCH // 2, step, 0)
    pltpu.sync_copy(out_v, out_hbm.at[pl.ds(base, b_per_w)])


def _granule_view(T, nt):
    th = nt * 128
    return (T[:th].T.reshape(4, 8, nt, 128).transpose(0, 2, 1, 3)
            .reshape(-1, 16))


@jax.jit
def kernel(SIDs, PIDs, P, Q, scientist_bias, paper_bias):
    B = SIDs.shape[0]
    b_per_w = B // NW
    sids = SIDs.astype(jnp.int32)
    pids = PIDs.astype(jnp.int32)
    pv = _granule_view(P, NT_P)
    qv = _granule_view(Q, NT_Q)
    pt = P[PTB:].reshape(-1, 128)
    qt = Q[QTB:].reshape(-1, 128)
    sb = scientist_bias.reshape(-1)
    pb = paper_bias.reshape(-1)

    mesh = plsc.VectorSubcoreMesh(core_axis_name="c", subcore_axis_name="s")
    f = pl.kernel(
        _mf_body,
        out_type=jax.ShapeDtypeStruct((B,), jnp.float32),
        mesh=mesh,
        compiler_params=pltpu.CompilerParams(
            needs_layout_passes=False, use_tc_tiling_on_sc=False),
        scratch_types=[
            pltpu.VMEM((b_per_w,), jnp.int32),
            pltpu.VMEM((b_per_w,), jnp.int32),
            pltpu.VMEM((2 * GR,), jnp.int32),
            pltpu.VMEM((2 * GR,), jnp.int32),
            pltpu.VMEM((CE,), jnp.int32),
            pltpu.VMEM((CE,), jnp.int32),
            pltpu.VMEM((2 * GR, 16), jnp.float32),
            pltpu.VMEM((2 * GR, 16), jnp.float32),
            pltpu.VMEM((CE, 128), jnp.float32),
            pltpu.VMEM((CE, 128), jnp.float32),
            pltpu.VMEM((b_per_w,), jnp.float32),
            pltpu.VMEM((b_per_w,), jnp.float32),
            pltpu.VMEM((b_per_w,), jnp.float32),
            pltpu.SemaphoreType.DMA,
            pltpu.SemaphoreType.DMA,
            pltpu.SemaphoreType.DMA,
            pltpu.SemaphoreType.DMA,
            pltpu.SemaphoreType.DMA,
            pltpu.SemaphoreType.DMA,
        ],
    )
    return f(sids, pids, pv, qv, pt, qt, sb, pb)
